# baseline (speedup 1.0000x reference)
import functools

import jax
import jax.numpy as jnp
from jax import lax
from jax.experimental import pallas as pl
from jax.experimental.pallas import tpu as pltpu
from jax.experimental.pallas import tpu_sc as plsc

N_CLASSES = 1024 + 3 * 128 + 1
DIM = 1024
BATCH = 4
SEQ = 2048
TOTAL = BATCH * SEQ

NC = 2
NS = 16
NW = NC * NS
LANES = 16
CHUNK = 16
POS_W = SEQ // NW
SUBC = POS_W // CHUNK
NCHUNK = BATCH * SUBC


def _sc_body(idx_hbm, table_hbm, pos_hbm, emb_out, mask_out,
             idx_v, mask_v, pbuf, rbuf0, rbuf1,
             isem, gsem0, gsem1, osem0, osem1):
    wid = lax.axis_index("s") * NC + lax.axis_index("c")
    pos0 = wid * POS_W

    rbufs = (rbuf0, rbuf1)
    gsems = (gsem0, gsem1)
    osems = (osem0, osem1)

    ph = pltpu.async_copy(pos_hbm.at[pl.ds(pos0, POS_W)], pbuf, isem)
    for b in range(BATCH):
        pltpu.sync_copy(idx_hbm.at[b, pl.ds(wid * SUBC, SUBC)],
                        idx_v.at[b])
    for b in range(BATCH):
        for s in range(SUBC):
            v = idx_v[b, s]
            mask_v[b, s] = jnp.where(v == N_CLASSES - 1, 1,
                                     0).astype(jnp.int32)
    for b in range(BATCH):
        pltpu.sync_copy(mask_v.at[b],
                        mask_out.at[b, pl.ds(wid * SUBC, SUBC)])

    ohandles = [None] * NCHUNK
    ghandles = [None] * NCHUNK

    def issue(c):
        slot = c % 2
        if c >= 2:
            ohandles[c - 2].wait()
        b, s = divmod(c, SUBC)
        ghandles[c] = pltpu.async_copy(
            table_hbm.at[idx_v.at[b, s]], rbufs[slot], gsems[slot])

    issue(0)
    ph.wait()
    for c in range(NCHUNK):
        if c + 1 < NCHUNK:
            issue(c + 1)
        slot = c % 2
        b, s = divmod(c, SUBC)
        ghandles[c].wait()
        rb = rbufs[slot]

        def addbody(r, carry):
            for jc in range(DIM // LANES):
                sl = pl.ds(jc * LANES, LANES)
                rb[r, sl] = rb[r, sl] + pbuf[s * CHUNK + r, sl]
            return carry

        lax.fori_loop(0, CHUNK, addbody, 0)
        row0 = b * SEQ + pos0 + s * CHUNK
        ohandles[c] = pltpu.async_copy(
            rb, emb_out.at[pl.ds(row0, CHUNK)], osems[slot])
    ohandles[NCHUNK - 2].wait()
    ohandles[NCHUNK - 1].wait()


@jax.jit
def _content_emb(idx3, embedding, pos2d):
    mesh = plsc.VectorSubcoreMesh(
        core_axis_name="c", subcore_axis_name="s",
        num_cores=NC, num_subcores=NS)
    run = pl.kernel(
        _sc_body,
        out_type=[
            jax.ShapeDtypeStruct((TOTAL, DIM), jnp.float32),
            jax.ShapeDtypeStruct((BATCH, SEQ // CHUNK, CHUNK), jnp.int32),
        ],
        mesh=mesh,
        scratch_types=[
            pltpu.VMEM((BATCH, SUBC, CHUNK), jnp.int32),
            pltpu.VMEM((BATCH, SUBC, CHUNK), jnp.int32),
            pltpu.VMEM((POS_W, DIM), jnp.float32),
            pltpu.VMEM((CHUNK, DIM), jnp.float32),
            pltpu.VMEM((CHUNK, DIM), jnp.float32),
            pltpu.SemaphoreType.DMA,
            pltpu.SemaphoreType.DMA,
            pltpu.SemaphoreType.DMA,
            pltpu.SemaphoreType.DMA,
            pltpu.SemaphoreType.DMA,
        ],
    )
    return run(idx3, embedding, pos2d)


def kernel(input, embedding, position_emb):
    emb_flat, mask3 = _content_emb(
        input.reshape(BATCH, SEQ // CHUNK, CHUNK), embedding,
        position_emb.reshape(SEQ, DIM))
    return (emb_flat.reshape(BATCH, SEQ, DIM),
            mask3.reshape(BATCH, SEQ))


# speedup vs baseline: 1.0144x; 1.0144x over previous
import functools

import jax
import jax.numpy as jnp
from jax import lax
from jax.experimental import pallas as pl
from jax.experimental.pallas import tpu as pltpu
from jax.experimental.pallas import tpu_sc as plsc

N_CLASSES = 1024 + 3 * 128 + 1
DIM = 1024
BATCH = 4
SEQ = 2048
TOTAL = BATCH * SEQ

NC = 2
NS = 16
NW = NC * NS
LANES = 16
CHUNK = 16
POS_W = SEQ // NW
SUBC = POS_W // CHUNK
NCHUNK = BATCH * SUBC


def _sc_body(idx_hbm, table_hbm, pos_hbm, emb_out, mask_out,
             idx_v, mask_v, pbuf, rbuf0, rbuf1, rbuf2,
             isem, psem, msem, gsem0, gsem1, gsem2,
             osem0, osem1, osem2):
    wid = lax.axis_index("s") * NC + lax.axis_index("c")
    pos0 = wid * POS_W

    rbufs = (rbuf0, rbuf1, rbuf2)
    gsems = (gsem0, gsem1, gsem2)
    osems = (osem0, osem1, osem2)
    NBUF = 3

    ih = pltpu.async_copy(
        idx_hbm.at[:, pl.ds(wid * SUBC, SUBC)], idx_v, isem)
    phandles = [
        pltpu.async_copy(
            pos_hbm.at[pl.ds(pos0 + s * CHUNK, CHUNK)],
            pbuf.at[pl.ds(s * CHUNK, CHUNK)], psem)
        for s in range(SUBC)
    ]

    ohandles = [None] * NCHUNK
    ghandles = [None] * NCHUNK

    def issue(c):
        slot = c % NBUF
        if c >= NBUF:
            ohandles[c - NBUF].wait()
        b, s = divmod(c, SUBC)
        ghandles[c] = pltpu.async_copy(
            table_hbm.at[idx_v.at[b, s]], rbufs[slot], gsems[slot])

    ih.wait()
    issue(0)
    issue(1)

    for b in range(BATCH):
        for s in range(SUBC):
            v = idx_v[b, s]
            mask_v[b, s] = jnp.where(v == N_CLASSES - 1, 1,
                                     0).astype(jnp.int32)
    mh = pltpu.async_copy(
        mask_v, mask_out.at[:, pl.ds(wid * SUBC, SUBC)], msem)

    for c in range(NCHUNK):
        if c + 2 < NCHUNK:
            issue(c + 2)
        slot = c % NBUF
        b, s = divmod(c, SUBC)
        if b == 0:
            phandles[s].wait()
        ghandles[c].wait()
        rb = rbufs[slot]

        def addbody(r, carry):
            for jc in range(DIM // LANES):
                sl = pl.ds(jc * LANES, LANES)
                rb[r, sl] = rb[r, sl] + pbuf[s * CHUNK + r, sl]
            return carry

        lax.fori_loop(0, CHUNK, addbody, 0)
        row0 = b * SEQ + pos0 + s * CHUNK
        ohandles[c] = pltpu.async_copy(
            rb, emb_out.at[pl.ds(row0, CHUNK)], osems[slot])
    mh.wait()
    ohandles[NCHUNK - 3].wait()
    ohandles[NCHUNK - 2].wait()
    ohandles[NCHUNK - 1].wait()


@jax.jit
def _content_emb(idx3, embedding, pos2d):
    mesh = plsc.VectorSubcoreMesh(
        core_axis_name="c", subcore_axis_name="s",
        num_cores=NC, num_subcores=NS)
    run = pl.kernel(
        _sc_body,
        out_type=[
            jax.ShapeDtypeStruct((TOTAL, DIM), jnp.float32),
            jax.ShapeDtypeStruct((BATCH, SEQ // CHUNK, CHUNK), jnp.int32),
        ],
        mesh=mesh,
        scratch_types=[
            pltpu.VMEM((BATCH, SUBC, CHUNK), jnp.int32),
            pltpu.VMEM((BATCH, SUBC, CHUNK), jnp.int32),
            pltpu.VMEM((POS_W, DIM), jnp.float32),
            pltpu.VMEM((CHUNK, DIM), jnp.float32),
            pltpu.VMEM((CHUNK, DIM), jnp.float32),
            pltpu.VMEM((CHUNK, DIM), jnp.float32),
            pltpu.SemaphoreType.DMA,
            pltpu.SemaphoreType.DMA,
            pltpu.SemaphoreType.DMA,
            pltpu.SemaphoreType.DMA,
            pltpu.SemaphoreType.DMA,
            pltpu.SemaphoreType.DMA,
            pltpu.SemaphoreType.DMA,
            pltpu.SemaphoreType.DMA,
            pltpu.SemaphoreType.DMA,
        ],
    )
    return run(idx3, embedding, pos2d)


def kernel(input, embedding, position_emb):
    emb_flat, mask3 = _content_emb(
        input.reshape(BATCH, SEQ // CHUNK, CHUNK), embedding,
        position_emb.reshape(SEQ, DIM))
    return (emb_flat.reshape(BATCH, SEQ, DIM),
            mask3.reshape(BATCH, SEQ))


# speedup vs baseline: 1.2494x; 1.2317x over previous
import jax
import jax.numpy as jnp
from jax import lax
from jax.experimental import pallas as pl
from jax.experimental.pallas import tpu as pltpu
from jax.experimental.pallas import tpu_sc as plsc

N_CLASSES = 1024 + 3 * 128 + 1
DIM = 1024
BATCH = 4
SEQ = 2048
TOTAL = BATCH * SEQ

NC = 2
NS = 16
NW = NC * NS
LANES = 16
PER_W = TOTAL // NW
CHUNK = 16
NCHUNK = PER_W // CHUNK
W_PER_B = SEQ // PER_W
NBUF = 3


def _sc_body(idx_hbm, table_hbm, pos_hbm, emb_out, mask_out,
             idx_v, mask_v, rbuf0, rbuf1, rbuf2, pbuf0, pbuf1, pbuf2,
             isem, msem, gsem0, gsem1, gsem2, psem0, psem1, psem2,
             osem0, osem1, osem2):
    wid = lax.axis_index("s") * NC + lax.axis_index("c")
    bw = lax.div(wid, W_PER_B)
    p0 = lax.rem(wid, W_PER_B) * PER_W

    rbufs = (rbuf0, rbuf1, rbuf2)
    pbufs = (pbuf0, pbuf1, pbuf2)
    gsems = (gsem0, gsem1, gsem2)
    psems = (psem0, psem1, psem2)
    osems = (osem0, osem1, osem2)

    pltpu.sync_copy(idx_hbm.at[bw, pl.ds(p0, PER_W)], idx_v)

    ohandles = [None] * NCHUNK
    ghandles = [None] * NCHUNK
    phandles = [None] * NCHUNK

    def issue(c):
        slot = c % NBUF
        if c >= NBUF:
            ohandles[c - NBUF].wait()
        ghandles[c] = pltpu.async_copy(
            table_hbm.at[idx_v.at[pl.ds(c * CHUNK, CHUNK)]],
            rbufs[slot], gsems[slot])
        phandles[c] = pltpu.async_copy(
            pos_hbm.at[0, pl.ds(p0 + c * CHUNK, CHUNK)],
            pbufs[slot], psems[slot])

    issue(0)
    issue(1)

    for k in range(PER_W // LANES):
        sl = pl.ds(k * LANES, LANES)
        mask_v[sl] = jnp.where(idx_v[sl] == N_CLASSES - 1, 1,
                               0).astype(jnp.int32)
    mh = pltpu.async_copy(mask_v, mask_out.at[bw, pl.ds(p0, PER_W)], msem)

    for c in range(NCHUNK):
        if c + 2 < NCHUNK:
            issue(c + 2)
        slot = c % NBUF
        ghandles[c].wait()
        phandles[c].wait()
        rb, pb = rbufs[slot], pbufs[slot]

        def addbody(r, carry):
            for jc in range(DIM // LANES):
                sl = pl.ds(jc * LANES, LANES)
                rb[r, sl] = rb[r, sl] + pb[r, sl]
            return carry

        lax.fori_loop(0, CHUNK, addbody, 0)
        ohandles[c] = pltpu.async_copy(
            rb, emb_out.at[bw, pl.ds(p0 + c * CHUNK, CHUNK)], osems[slot])
    mh.wait()
    for c in range(NCHUNK - NBUF, NCHUNK):
        ohandles[c].wait()


@jax.jit
def _content_emb(idx, embedding, position_emb):
    mesh = plsc.VectorSubcoreMesh(
        core_axis_name="c", subcore_axis_name="s",
        num_cores=NC, num_subcores=NS)
    run = pl.kernel(
        _sc_body,
        out_type=[
            jax.ShapeDtypeStruct((BATCH, SEQ, DIM), jnp.float32),
            jax.ShapeDtypeStruct((BATCH, SEQ), jnp.int32),
        ],
        mesh=mesh,
        scratch_types=[
            pltpu.VMEM((PER_W,), jnp.int32),
            pltpu.VMEM((PER_W,), jnp.int32),
            pltpu.VMEM((CHUNK, DIM), jnp.float32),
            pltpu.VMEM((CHUNK, DIM), jnp.float32),
            pltpu.VMEM((CHUNK, DIM), jnp.float32),
            pltpu.VMEM((CHUNK, DIM), jnp.float32),
            pltpu.VMEM((CHUNK, DIM), jnp.float32),
            pltpu.VMEM((CHUNK, DIM), jnp.float32),
            pltpu.SemaphoreType.DMA,
            pltpu.SemaphoreType.DMA,
            pltpu.SemaphoreType.DMA,
            pltpu.SemaphoreType.DMA,
            pltpu.SemaphoreType.DMA,
            pltpu.SemaphoreType.DMA,
            pltpu.SemaphoreType.DMA,
            pltpu.SemaphoreType.DMA,
            pltpu.SemaphoreType.DMA,
            pltpu.SemaphoreType.DMA,
            pltpu.SemaphoreType.DMA,
        ],
    )
    return run(idx, embedding, position_emb)


def kernel(input, embedding, position_emb):
    emb, mask = _content_emb(input, embedding, position_emb)
    return (emb, mask)


# speedup vs baseline: 1.3620x; 1.0901x over previous
import jax
import jax.numpy as jnp
from jax import lax
from jax.experimental import pallas as pl
from jax.experimental.pallas import tpu as pltpu
from jax.experimental.pallas import tpu_sc as plsc

N_CLASSES = 1024 + 3 * 128 + 1
DIM = 1024
BATCH = 4
SEQ = 2048
TOTAL = BATCH * SEQ

NC = 2
NS = 16
NW = NC * NS
LANES = 16
CHUNK = 16
BLK = 128
NSTEP = BLK // CHUNK
NBLK = SEQ // BLK
BPAIR = 2
NRING = 2


def _sc_body(idx_hbm, table_hbm, pos_hbm, emb_out, mask_out,
             idx_v, mask_v, pbuf0, pbuf1,
             rbuf00, rbuf01, rbuf10, rbuf11,
             isem, msem, psem0, psem1,
             gsem00, gsem01, gsem10, gsem11,
             osem00, osem01, osem10, osem11):
    wid = lax.axis_index("s") * NC + lax.axis_index("c")
    blk = lax.rem(wid, NBLK)
    b0 = lax.div(wid, NBLK) * BPAIR
    p0 = blk * BLK

    pbufs = (pbuf0, pbuf1)
    psems = (psem0, psem1)
    rbufs = ((rbuf00, rbuf01), (rbuf10, rbuf11))
    gsems = ((gsem00, gsem01), (gsem10, gsem11))
    osems = ((osem00, osem01), (osem10, osem11))

    pltpu.sync_copy(idx_hbm.at[pl.ds(b0, BPAIR), pl.ds(p0, BLK)], idx_v)

    ohandles = [[None, None] for _ in range(NSTEP)]
    ghandles = [[None, None] for _ in range(NSTEP)]
    phandles = [None] * NSTEP

    def issue(s):
        slot = s % NRING
        if s >= NRING:
            ohandles[s - NRING][0].wait()
            ohandles[s - NRING][1].wait()
        phandles[s] = pltpu.async_copy(
            pos_hbm.at[0, pl.ds(p0 + s * CHUNK, CHUNK)],
            pbufs[slot], psems[slot])
        for j in range(BPAIR):
            ghandles[s][j] = pltpu.async_copy(
                table_hbm.at[idx_v.at[j, pl.ds(s * CHUNK, CHUNK)]],
                rbufs[slot][j], gsems[slot][j])

    issue(0)

    for j in range(BPAIR):
        for k in range(BLK // LANES):
            sl = pl.ds(k * LANES, LANES)
            mask_v[j, sl] = jnp.where(idx_v[j, sl] == N_CLASSES - 1, 1,
                                      0).astype(jnp.int32)
    mh = pltpu.async_copy(
        mask_v, mask_out.at[pl.ds(b0, BPAIR), pl.ds(p0, BLK)], msem)

    for s in range(NSTEP):
        if s + 1 < NSTEP:
            issue(s + 1)
        slot = s % NRING
        phandles[s].wait()
        pb = pbufs[slot]
        for j in range(BPAIR):
            ghandles[s][j].wait()
            rb = rbufs[slot][j]

            def addbody(r, carry):
                for jc in range(DIM // LANES):
                    sl = pl.ds(jc * LANES, LANES)
                    rb[r, sl] = rb[r, sl] + pb[r, sl]
                return carry

            lax.fori_loop(0, CHUNK, addbody, 0)
            ohandles[s][j] = pltpu.async_copy(
                rb, emb_out.at[b0 + j, pl.ds(p0 + s * CHUNK, CHUNK)],
                osems[slot][j])
    mh.wait()
    for s in range(NSTEP - NRING, NSTEP):
        ohandles[s][0].wait()
        ohandles[s][1].wait()


@jax.jit
def _content_emb(idx, embedding, position_emb):
    mesh = plsc.VectorSubcoreMesh(
        core_axis_name="c", subcore_axis_name="s",
        num_cores=NC, num_subcores=NS)
    run = pl.kernel(
        _sc_body,
        out_type=[
            jax.ShapeDtypeStruct((BATCH, SEQ, DIM), jnp.float32),
            jax.ShapeDtypeStruct((BATCH, SEQ), jnp.int32),
        ],
        mesh=mesh,
        scratch_types=[
            pltpu.VMEM((BPAIR, BLK), jnp.int32),
            pltpu.VMEM((BPAIR, BLK), jnp.int32),
            pltpu.VMEM((CHUNK, DIM), jnp.float32),
            pltpu.VMEM((CHUNK, DIM), jnp.float32),
            pltpu.VMEM((CHUNK, DIM), jnp.float32),
            pltpu.VMEM((CHUNK, DIM), jnp.float32),
            pltpu.VMEM((CHUNK, DIM), jnp.float32),
            pltpu.VMEM((CHUNK, DIM), jnp.float32),
            pltpu.SemaphoreType.DMA,
            pltpu.SemaphoreType.DMA,
            pltpu.SemaphoreType.DMA,
            pltpu.SemaphoreType.DMA,
            pltpu.SemaphoreType.DMA,
            pltpu.SemaphoreType.DMA,
            pltpu.SemaphoreType.DMA,
            pltpu.SemaphoreType.DMA,
            pltpu.SemaphoreType.DMA,
            pltpu.SemaphoreType.DMA,
            pltpu.SemaphoreType.DMA,
            pltpu.SemaphoreType.DMA,
        ],
    )
    return run(idx, embedding, position_emb)


def kernel(input, embedding, position_emb):
    emb, mask = _content_emb(input, embedding, position_emb)
    return (emb, mask)


# speedup vs baseline: 1.4713x; 1.0802x over previous
import jax
import jax.numpy as jnp
from jax import lax
from jax.experimental import pallas as pl
from jax.experimental.pallas import tpu as pltpu
from jax.experimental.pallas import tpu_sc as plsc

N_CLASSES = 1024 + 3 * 128 + 1
DIM = 1024
BATCH = 4
SEQ = 2048
TOTAL = BATCH * SEQ

NC = 2
NS = 16
NW = NC * NS
LANES = 16
CHUNK = 16
BLK = 128
NSTEP = BLK // CHUNK
NBLK = SEQ // BLK
BPAIR = 2
NRING = 2


def _sc_body(idx_hbm, table_hbm, pos_hbm, emb_out, mask_out,
             idx_v, mask_v, pbuf0, pbuf1,
             rbuf00, rbuf01, rbuf10, rbuf11,
             isem, msem, psem0, psem1,
             gsem00, gsem01, gsem10, gsem11,
             osem00, osem01, osem10, osem11):
    wid = lax.axis_index("s") * NC + lax.axis_index("c")
    blk = lax.rem(wid, NBLK)
    b0 = lax.div(wid, NBLK) * BPAIR
    p0 = blk * BLK

    pbufs = (pbuf0, pbuf1)
    psems = (psem0, psem1)
    rbufs = ((rbuf00, rbuf01), (rbuf10, rbuf11))
    gsems = ((gsem00, gsem01), (gsem10, gsem11))
    osems = ((osem00, osem01), (osem10, osem11))

    pltpu.sync_copy(idx_hbm.at[pl.ds(b0, BPAIR), pl.ds(p0, BLK)], idx_v)

    ohandles = [[None, None] for _ in range(NSTEP)]
    ghandles = [[None, None] for _ in range(NSTEP)]
    phandles = [None] * NSTEP

    def issue(s):
        slot = s % NRING
        if s >= NRING:
            ohandles[s - NRING][0].wait()
            ohandles[s - NRING][1].wait()
        for j in range(BPAIR):
            ghandles[s][j] = pltpu.async_copy(
                table_hbm.at[idx_v.at[j, pl.ds(s * CHUNK, CHUNK)]],
                rbufs[slot][j], gsems[slot][j])
        phandles[s] = pltpu.async_copy(
            pos_hbm.at[0, pl.ds(p0 + s * CHUNK, CHUNK)],
            pbufs[slot], psems[slot])

    issue(0)

    for j in range(BPAIR):
        for k in range(BLK // LANES):
            sl = pl.ds(k * LANES, LANES)
            mask_v[j, sl] = jnp.where(idx_v[j, sl] == N_CLASSES - 1, 1,
                                      0).astype(jnp.int32)
    mh = pltpu.async_copy(
        mask_v, mask_out.at[pl.ds(b0, BPAIR), pl.ds(p0, BLK)], msem)

    for s in range(NSTEP):
        if s + 1 < NSTEP:
            issue(s + 1)
        slot = s % NRING
        phandles[s].wait()
        ghandles[s][0].wait()
        ghandles[s][1].wait()
        pb = pbufs[slot]
        rb0, rb1 = rbufs[slot]

        def addbody(r, carry):
            for jc in range(DIM // LANES):
                sl = pl.ds(jc * LANES, LANES)
                v = pb[r, sl]
                rb0[r, sl] = rb0[r, sl] + v
                rb1[r, sl] = rb1[r, sl] + v
            return carry

        lax.fori_loop(0, CHUNK, addbody, 0)
        for j in range(BPAIR):
            ohandles[s][j] = pltpu.async_copy(
                rbufs[slot][j],
                emb_out.at[b0 + j, pl.ds(p0 + s * CHUNK, CHUNK)],
                osems[slot][j])
    mh.wait()
    for s in range(NSTEP - NRING, NSTEP):
        ohandles[s][0].wait()
        ohandles[s][1].wait()


@jax.jit
def _content_emb(idx, embedding, position_emb):
    mesh = plsc.VectorSubcoreMesh(
        core_axis_name="c", subcore_axis_name="s",
        num_cores=NC, num_subcores=NS)
    run = pl.kernel(
        _sc_body,
        out_type=[
            jax.ShapeDtypeStruct((BATCH, SEQ, DIM), jnp.float32),
            jax.ShapeDtypeStruct((BATCH, SEQ), jnp.int32),
        ],
        mesh=mesh,
        scratch_types=[
            pltpu.VMEM((BPAIR, BLK), jnp.int32),
            pltpu.VMEM((BPAIR, BLK), jnp.int32),
            pltpu.VMEM((CHUNK, DIM), jnp.float32),
            pltpu.VMEM((CHUNK, DIM), jnp.float32),
            pltpu.VMEM((CHUNK, DIM), jnp.float32),
            pltpu.VMEM((CHUNK, DIM), jnp.float32),
            pltpu.VMEM((CHUNK, DIM), jnp.float32),
            pltpu.VMEM((CHUNK, DIM), jnp.float32),
            pltpu.SemaphoreType.DMA,
            pltpu.SemaphoreType.DMA,
            pltpu.SemaphoreType.DMA,
            pltpu.SemaphoreType.DMA,
            pltpu.SemaphoreType.DMA,
            pltpu.SemaphoreType.DMA,
            pltpu.SemaphoreType.DMA,
            pltpu.SemaphoreType.DMA,
            pltpu.SemaphoreType.DMA,
            pltpu.SemaphoreType.DMA,
            pltpu.SemaphoreType.DMA,
            pltpu.SemaphoreType.DMA,
        ],
    )
    return run(idx, embedding, position_emb)


def kernel(input, embedding, position_emb):
    emb, mask = _content_emb(input, embedding, position_emb)
    return (emb, mask)


# speedup vs baseline: 1.5184x; 1.0320x over previous
import jax
import jax.numpy as jnp
from jax import lax
from jax.experimental import pallas as pl
from jax.experimental.pallas import tpu as pltpu
from jax.experimental.pallas import tpu_sc as plsc

N_CLASSES = 1024 + 3 * 128 + 1
DIM = 1024
HD = DIM // 2
BATCH = 4
SEQ = 2048
TOTAL = BATCH * SEQ

NC = 2
NS = 16
NW = NC * NS
LANES = 16
CHUNK = 16
BLK = SEQ // NW
NPC = BLK // CHUNK
NSTEP = NPC * 2
NRING = 2


def _sc_body(idx_hbm, table_hbm, pos_hbm, emb_out, mask_out,
             idx_v, mask_v, pbuf0, pbuf1,
             rb00, rb01, rb02, rb03, rb10, rb11, rb12, rb13,
             isem, msem, psem0, psem1,
             gs00, gs01, gs02, gs03, gs10, gs11, gs12, gs13,
             os00, os01, os02, os03, os10, os11, os12, os13):
    wid = lax.axis_index("s") * NC + lax.axis_index("c")
    p0 = wid * BLK

    pbufs = (pbuf0, pbuf1)
    psems = (psem0, psem1)
    rbufs = ((rb00, rb01, rb02, rb03), (rb10, rb11, rb12, rb13))
    gsems = ((gs00, gs01, gs02, gs03), (gs10, gs11, gs12, gs13))
    osems = ((os00, os01, os02, os03), (os10, os11, os12, os13))

    ihandles = [pltpu.async_copy(idx_hbm.at[j, pl.ds(p0, BLK)],
                                 idx_v.at[j], isem)
                for j in range(BATCH)]
    for h in ihandles:
        h.wait()

    ohandles = [[None] * BATCH for _ in range(NSTEP)]
    ghandles = [[None] * BATCH for _ in range(NSTEP)]
    phandles = [None] * NSTEP

    def issue(s):
        slot = s % NRING
        if s >= NRING:
            for j in range(BATCH):
                ohandles[s - NRING][j].wait()
        pc, h = divmod(s, 2)
        for j in range(BATCH):
            ghandles[s][j] = pltpu.async_copy(
                table_hbm.at[idx_v.at[j, pl.ds(pc * CHUNK, CHUNK)],
                             pl.ds(h * HD, HD)],
                rbufs[slot][j], gsems[slot][j])
        phandles[s] = pltpu.async_copy(
            pos_hbm.at[0, pl.ds(p0 + pc * CHUNK, CHUNK),
                       pl.ds(h * HD, HD)],
            pbufs[slot], psems[slot])

    issue(0)

    for j in range(BATCH):
        for k in range(BLK // LANES):
            sl = pl.ds(k * LANES, LANES)
            mask_v[j, sl] = jnp.where(idx_v[j, sl] == N_CLASSES - 1, 1,
                                      0).astype(jnp.int32)
    mhandles = [pltpu.async_copy(mask_v.at[j],
                                 mask_out.at[j, pl.ds(p0, BLK)], msem)
                for j in range(BATCH)]

    for s in range(NSTEP):
        if s + 1 < NSTEP:
            issue(s + 1)
        slot = s % NRING
        pc, h = divmod(s, 2)
        phandles[s].wait()
        for j in range(BATCH):
            ghandles[s][j].wait()
        pb = pbufs[slot]
        rb0, rb1, rb2, rb3 = rbufs[slot]

        def addbody(r, carry):
            for jc in range(HD // LANES):
                sl = pl.ds(jc * LANES, LANES)
                v = pb[r, sl]
                rb0[r, sl] = rb0[r, sl] + v
                rb1[r, sl] = rb1[r, sl] + v
                rb2[r, sl] = rb2[r, sl] + v
                rb3[r, sl] = rb3[r, sl] + v
            return carry

        lax.fori_loop(0, CHUNK, addbody, 0)
        for j in range(BATCH):
            ohandles[s][j] = pltpu.async_copy(
                rbufs[slot][j],
                emb_out.at[j, pl.ds(p0 + pc * CHUNK, CHUNK),
                           pl.ds(h * HD, HD)],
                osems[slot][j])
    for h in mhandles:
        h.wait()
    for s in range(NSTEP - NRING, NSTEP):
        for j in range(BATCH):
            ohandles[s][j].wait()


@jax.jit
def _content_emb(idx, embedding, position_emb):
    mesh = plsc.VectorSubcoreMesh(
        core_axis_name="c", subcore_axis_name="s",
        num_cores=NC, num_subcores=NS)
    run = pl.kernel(
        _sc_body,
        out_type=[
            jax.ShapeDtypeStruct((BATCH, SEQ, DIM), jnp.float32),
            jax.ShapeDtypeStruct((BATCH, SEQ), jnp.int32),
        ],
        mesh=mesh,
        scratch_types=(
            [pltpu.VMEM((BATCH, BLK), jnp.int32)] * 2 +
            [pltpu.VMEM((CHUNK, HD), jnp.float32)] * 10 +
            [pltpu.SemaphoreType.DMA] * 20
        ),
    )
    return run(idx, embedding, position_emb)


def kernel(input, embedding, position_emb):
    emb, mask = _content_emb(input, embedding, position_emb)
    return (emb, mask)


# speedup vs baseline: 1.5426x; 1.0160x over previous
import jax
import jax.numpy as jnp
from jax import lax
from jax.experimental import pallas as pl
from jax.experimental.pallas import tpu as pltpu
from jax.experimental.pallas import tpu_sc as plsc

N_CLASSES = 1024 + 3 * 128 + 1
DIM = 1024
HD = DIM // 2
BATCH = 4
SEQ = 2048
TOTAL = BATCH * SEQ

NC = 2
NS = 16
NW = NC * NS
LANES = 16
CHUNK = 16
BLK = SEQ // NW
NPC = BLK // CHUNK
NSTEP = NPC * 2
NRING = 3


def _sc_body(idx_hbm, table_hbm, pos_hbm, emb_out, mask_out,
             idx_v, mask_v, pbuf0, pbuf1, pbuf2,
             rb00, rb01, rb02, rb03, rb10, rb11, rb12, rb13,
             rb20, rb21, rb22, rb23,
             isem, msem, psem0, psem1, psem2,
             gs00, gs01, gs02, gs03, gs10, gs11, gs12, gs13,
             gs20, gs21, gs22, gs23,
             os00, os01, os02, os03, os10, os11, os12, os13,
             os20, os21, os22, os23):
    wid = lax.axis_index("s") * NC + lax.axis_index("c")
    p0 = wid * BLK

    pbufs = (pbuf0, pbuf1, pbuf2)
    psems = (psem0, psem1, psem2)
    rbufs = ((rb00, rb01, rb02, rb03), (rb10, rb11, rb12, rb13),
             (rb20, rb21, rb22, rb23))
    gsems = ((gs00, gs01, gs02, gs03), (gs10, gs11, gs12, gs13),
             (gs20, gs21, gs22, gs23))
    osems = ((os00, os01, os02, os03), (os10, os11, os12, os13),
             (os20, os21, os22, os23))

    ihandles = [pltpu.async_copy(idx_hbm.at[j, pl.ds(p0, BLK)],
                                 idx_v.at[j], isem)
                for j in range(BATCH)]
    for h in ihandles:
        h.wait()

    ohandles = [[None] * BATCH for _ in range(NSTEP)]
    ghandles = [[None] * BATCH for _ in range(NSTEP)]
    phandles = [None] * NSTEP

    def issue(s):
        slot = s % NRING
        if s >= NRING:
            for j in range(BATCH):
                ohandles[s - NRING][j].wait()
        pc, h = divmod(s, 2)
        for j in range(BATCH):
            ghandles[s][j] = pltpu.async_copy(
                table_hbm.at[idx_v.at[j, pl.ds(pc * CHUNK, CHUNK)],
                             pl.ds(h * HD, HD)],
                rbufs[slot][j], gsems[slot][j])
        phandles[s] = pltpu.async_copy(
            pos_hbm.at[0, pl.ds(p0 + pc * CHUNK, CHUNK),
                       pl.ds(h * HD, HD)],
            pbufs[slot], psems[slot])

    issue(0)
    issue(1)

    for j in range(BATCH):
        for k in range(BLK // LANES):
            sl = pl.ds(k * LANES, LANES)
            mask_v[j, sl] = jnp.where(idx_v[j, sl] == N_CLASSES - 1, 1,
                                      0).astype(jnp.int32)
    mhandles = [pltpu.async_copy(mask_v.at[j],
                                 mask_out.at[j, pl.ds(p0, BLK)], msem)
                for j in range(BATCH)]

    for s in range(NSTEP):
        if s + 2 < NSTEP:
            issue(s + 2)
        slot = s % NRING
        pc, h = divmod(s, 2)
        phandles[s].wait()
        for j in range(BATCH):
            ghandles[s][j].wait()
        pb = pbufs[slot]
        rb0, rb1, rb2, rb3 = rbufs[slot]

        def addbody(r, carry):
            for jc in range(HD // LANES):
                sl = pl.ds(jc * LANES, LANES)
                v = pb[r, sl]
                rb0[r, sl] = rb0[r, sl] + v
                rb1[r, sl] = rb1[r, sl] + v
                rb2[r, sl] = rb2[r, sl] + v
                rb3[r, sl] = rb3[r, sl] + v
            return carry

        lax.fori_loop(0, CHUNK, addbody, 0)
        for j in range(BATCH):
            ohandles[s][j] = pltpu.async_copy(
                rbufs[slot][j],
                emb_out.at[j, pl.ds(p0 + pc * CHUNK, CHUNK),
                           pl.ds(h * HD, HD)],
                osems[slot][j])
    for h in mhandles:
        h.wait()
    for s in range(NSTEP - NRING, NSTEP):
        for j in range(BATCH):
            ohandles[s][j].wait()


@jax.jit
def _content_emb(idx, embedding, position_emb):
    mesh = plsc.VectorSubcoreMesh(
        core_axis_name="c", subcore_axis_name="s",
        num_cores=NC, num_subcores=NS)
    run = pl.kernel(
        _sc_body,
        out_type=[
            jax.ShapeDtypeStruct((BATCH, SEQ, DIM), jnp.float32),
            jax.ShapeDtypeStruct((BATCH, SEQ), jnp.int32),
        ],
        mesh=mesh,
        scratch_types=(
            [pltpu.VMEM((BATCH, BLK), jnp.int32)] * 2 +
            [pltpu.VMEM((CHUNK, HD), jnp.float32)] * 15 +
            [pltpu.SemaphoreType.DMA] * 29
        ),
    )
    return run(idx, embedding, position_emb)


def kernel(input, embedding, position_emb):
    emb, mask = _content_emb(input, embedding, position_emb)
    return (emb, mask)
